# baseline (speedup 1.0000x reference)
import functools

import jax
import jax.numpy as jnp
from jax import lax
from jax.experimental import pallas as pl
from jax.experimental.pallas import tpu as pltpu
from jax.experimental.pallas import tpu_sc as plsc

N_NODES = 10000
N_EDGES = 320000
D = 128

NC = 2
NS = 16
NW = NC * NS
EPW = N_EDGES // NW
CH = 80
EPWP = 10080
NCH = EPWP // CH
NCHH = NCH // 2
NPAD = 10240
DSL = NPAD // NS
NR = 10240
RSL = NR // NS

_MESH = plsc.VectorSubcoreMesh(core_axis_name="c", subcore_axis_name="s")


def _deg_body(dst_hbm, out_hbm, idx_v, ones_v, zero_v, deg_sh, sem):
    del sem
    cid = lax.axis_index("c")
    sid = lax.axis_index("s")
    wid = sid * NC + cid
    pltpu.sync_copy(dst_hbm.at[wid], idx_v)
    for i in range(CH // 16):
        ones_v[pl.ds(i * 16, 16)] = jnp.full((16,), 1.0, jnp.float32)
    for i in range(DSL // 16):
        zero_v[pl.ds(i * 16, 16)] = jnp.zeros((16,), jnp.float32)
    pltpu.sync_copy(zero_v, deg_sh.at[pl.ds(sid * DSL, DSL)])
    plsc.subcore_barrier()

    @pl.loop(0, NCH)
    def _chunk(j):
        pltpu.sync_copy(ones_v, deg_sh.at[idx_v.at[j]], add=True)

    plsc.subcore_barrier()
    pltpu.sync_copy(deg_sh.at[pl.ds(sid * DSL, DSL)],
                    out_hbm.at[cid, pl.ds(sid * DSL, DSL)])


@functools.partial(
    pl.kernel,
    out_type=jax.ShapeDtypeStruct((NC, NPAD), jnp.float32),
    mesh=_MESH,
    scratch_types=[
        pltpu.VMEM((NCH, CH), jnp.int32),
        pltpu.VMEM((CH,), jnp.float32),
        pltpu.VMEM((DSL,), jnp.float32),
        pltpu.VMEM_SHARED((NPAD,), jnp.float32),
        pltpu.SemaphoreType.DMA,
    ],
)
def _deg_kernel(dst_hbm, out_hbm, idx_v, ones_v, zero_v, deg_sh, sem):
    _deg_body(dst_hbm, out_hbm, idx_v, ones_v, zero_v, deg_sh, sem)


def _scatter_body(g_hbm, src_hbm, dst_hbm, zeros_hbm, out_hbm,
                  sidx_v, didx_v, rows, acc_sh, gsems, ssems):
    cid = lax.axis_index("c")
    sid = lax.axis_index("s")
    wid = sid * NC + cid
    pltpu.sync_copy(zeros_hbm.at[pl.ds(sid * RSL, RSL)],
                    acc_sh.at[pl.ds(sid * RSL, RSL)])
    plsc.subcore_barrier()

    def gather(j, k):
        pltpu.async_copy(g_hbm.at[sidx_v.at[j]], rows[k], gsems[k])

    def gwait(j, k):
        pltpu.make_async_copy(g_hbm.at[sidx_v.at[j]], rows[k],
                              gsems[k]).wait()

    def scat(j, k):
        pltpu.async_copy(rows[k], acc_sh.at[didx_v.at[j]], ssems[k],
                         add=True)

    def swait(j, k):
        pltpu.make_async_copy(rows[k], acc_sh.at[didx_v.at[j]],
                              ssems[k]).wait()

    for hp in range(2):
        pltpu.sync_copy(src_hbm.at[wid, hp], sidx_v)
        pltpu.sync_copy(dst_hbm.at[wid, hp], didx_v)
        for k in range(3):
            gather(k, k)
        for k in range(3):
            gwait(k, k)
            scat(k, k)
            gather(k + 3, k)

        @pl.loop(1, NCHH // 3 - 1)
        def _triple(p):
            j = 3 * p
            for k in range(3):
                swait(j + k - 3, k)
                gwait(j + k, k)
                scat(j + k, k)
                gather(j + k + 3, k)

        jt = NCHH - 3
        for k in range(3):
            swait(jt + k - 3, k)
            gwait(jt + k, k)
            scat(jt + k, k)
        for k in range(3):
            swait(jt + k, k)

    plsc.subcore_barrier()
    pltpu.sync_copy(acc_sh.at[pl.ds(sid * RSL, RSL)],
                    out_hbm.at[cid, pl.ds(sid * RSL, RSL)])


@functools.partial(
    pl.kernel,
    out_type=jax.ShapeDtypeStruct((NC, NR, D), jnp.float32),
    mesh=_MESH,
    scratch_types=[
        pltpu.VMEM((NCHH, CH), jnp.int32),
        pltpu.VMEM((NCHH, CH), jnp.int32),
        pltpu.VMEM((CH, D), jnp.float32),
        pltpu.VMEM((CH, D), jnp.float32),
        pltpu.VMEM((CH, D), jnp.float32),
        pltpu.VMEM_SHARED((NR, D), jnp.float32),
        pltpu.SemaphoreType.DMA,
        pltpu.SemaphoreType.DMA,
        pltpu.SemaphoreType.DMA,
        pltpu.SemaphoreType.DMA,
        pltpu.SemaphoreType.DMA,
        pltpu.SemaphoreType.DMA,
    ],
)
def _scatter_kernel(g_hbm, src_hbm, dst_hbm, zeros_hbm, out_hbm,
                    sidx_v, didx_v, r0, r1, r2, acc_sh,
                    gs0, gs1, gs2, ss0, ss1, ss2):
    _scatter_body(g_hbm, src_hbm, dst_hbm, zeros_hbm, out_hbm,
                  sidx_v, didx_v, (r0, r1, r2), acc_sh, (gs0, gs1, gs2),
                  (ss0, ss1, ss2))


RB = 400
NRB = N_NODES // RB


def _dis(degp_ref):
    return lax.rsqrt(degp_ref[0] + degp_ref[1] + 1.0)


def _pre_body(x_ref, w_ref, degp_ref, o_ref):
    h = jnp.dot(x_ref[...], w_ref[...], preferred_element_type=jnp.float32)
    o_ref[...] = h * _dis(degp_ref)


def _pre_kernel(x, w, degp):
    return pl.pallas_call(
        _pre_body,
        grid=(NRB,),
        in_specs=[
            pl.BlockSpec((RB, D), lambda i: (i, 0)),
            pl.BlockSpec((D, D), lambda i: (0, 0)),
            pl.BlockSpec((NC, RB, 1), lambda i: (0, i, 0)),
        ],
        out_specs=pl.BlockSpec((RB, D), lambda i: (i, 0)),
        out_shape=jax.ShapeDtypeStruct((N_NODES, D), jnp.float32),
    )(x, w, degp)


def _mid_body(t_ref, g_ref, degp_ref, b_ref, w_ref, o_ref):
    dis = _dis(degp_ref)
    tsum = t_ref[0] + t_ref[1] + g_ref[...]
    h = jnp.maximum(tsum * dis + b_ref[...], 0.0)
    o_ref[...] = jnp.dot(h, w_ref[...],
                         preferred_element_type=jnp.float32) * dis


def _mid_kernel(t, g, degp, b, w):
    return pl.pallas_call(
        _mid_body,
        grid=(NRB,),
        in_specs=[
            pl.BlockSpec((NC, RB, D), lambda i: (0, i, 0)),
            pl.BlockSpec((RB, D), lambda i: (i, 0)),
            pl.BlockSpec((NC, RB, 1), lambda i: (0, i, 0)),
            pl.BlockSpec((1, D), lambda i: (0, 0)),
            pl.BlockSpec((D, D), lambda i: (0, 0)),
        ],
        out_specs=pl.BlockSpec((RB, D), lambda i: (i, 0)),
        out_shape=jax.ShapeDtypeStruct((N_NODES, D), jnp.float32),
    )(t, g, degp, b, w)


def _fin_body(t_ref, g_ref, degp_ref, b_ref, o_ref):
    tsum = t_ref[0] + t_ref[1] + g_ref[...]
    o_ref[...] = tsum * _dis(degp_ref) + b_ref[...]


def _fin_kernel(t, g, degp, b):
    return pl.pallas_call(
        _fin_body,
        grid=(NRB,),
        in_specs=[
            pl.BlockSpec((NC, RB, D), lambda i: (0, i, 0)),
            pl.BlockSpec((RB, D), lambda i: (i, 0)),
            pl.BlockSpec((NC, RB, 1), lambda i: (0, i, 0)),
            pl.BlockSpec((1, D), lambda i: (0, 0)),
        ],
        out_specs=pl.BlockSpec((RB, D), lambda i: (i, 0)),
        out_shape=jax.ShapeDtypeStruct((N_NODES, D), jnp.float32),
    )(t, g, degp, b)


def kernel(x, edge_index, W1, b1, W2, b2):
    ei = edge_index.astype(jnp.int32)
    npad = EPWP - EPW
    pad_src = jnp.broadcast_to((jnp.arange(npad, dtype=jnp.int32) * 41)
                               % N_NODES, (NW, npad))
    pad_dst = jnp.broadcast_to(N_NODES + jnp.arange(npad, dtype=jnp.int32)
                               % (NR - N_NODES), (NW, npad))
    src4 = jnp.concatenate([ei[0].reshape(NW, EPW), pad_src],
                           axis=1).reshape(NW, 2, NCHH, CH)
    dst4 = jnp.concatenate([ei[1].reshape(NW, EPW), pad_dst],
                           axis=1).reshape(NW, 2, NCHH, CH)
    dst3 = dst4.reshape(NW, NCH, CH)
    zeros = jnp.zeros((NR, D), jnp.float32)
    b1r = b1.reshape(1, D)
    b2r = b2.reshape(1, D)

    degp = _deg_kernel(dst3).reshape(NC, NPAD, 1)

    g1 = _pre_kernel(x, W1, degp)
    t1 = _scatter_kernel(g1, src4, dst4, zeros)
    g2 = _mid_kernel(t1, g1, degp, b1r, W2)
    t2 = _scatter_kernel(g2, src4, dst4, zeros)
    out = _fin_kernel(t2, g2, degp, b2r)
    return out


# speedup vs baseline: 1.1011x; 1.1011x over previous
import functools

import jax
import jax.numpy as jnp
from jax import lax
from jax.experimental import pallas as pl
from jax.experimental.pallas import tpu as pltpu
from jax.experimental.pallas import tpu_sc as plsc

N_NODES = 10000
N_EDGES = 320000
D = 128

NC = 2
NS = 16
NW = NC * NS
EPW = N_EDGES // NW
CH = 128
EPWP = 10240
NCH = EPWP // CH
NCHH = NCH // 2
NPAD = 10240
DSL = NPAD // NS
NR = 10240
RSL = NR // NS

_MESH = plsc.VectorSubcoreMesh(core_axis_name="c", subcore_axis_name="s")


def _deg_body(dst_hbm, out_hbm, idx_v, ones_v, zero_v, deg_sh, sem):
    del sem
    cid = lax.axis_index("c")
    sid = lax.axis_index("s")
    wid = sid * NC + cid
    pltpu.sync_copy(dst_hbm.at[wid], idx_v)
    for i in range(CH // 16):
        ones_v[pl.ds(i * 16, 16)] = jnp.full((16,), 1.0, jnp.float32)
    for i in range(DSL // 16):
        zero_v[pl.ds(i * 16, 16)] = jnp.zeros((16,), jnp.float32)
    pltpu.sync_copy(zero_v, deg_sh.at[pl.ds(sid * DSL, DSL)])
    plsc.subcore_barrier()

    @pl.loop(0, NCH)
    def _chunk(j):
        pltpu.sync_copy(ones_v, deg_sh.at[idx_v.at[j]], add=True)

    plsc.subcore_barrier()
    pltpu.sync_copy(deg_sh.at[pl.ds(sid * DSL, DSL)],
                    out_hbm.at[cid, pl.ds(sid * DSL, DSL)])


@functools.partial(
    pl.kernel,
    out_type=jax.ShapeDtypeStruct((NC, NPAD), jnp.float32),
    mesh=_MESH,
    scratch_types=[
        pltpu.VMEM((NCH, CH), jnp.int32),
        pltpu.VMEM((CH,), jnp.float32),
        pltpu.VMEM((DSL,), jnp.float32),
        pltpu.VMEM_SHARED((NPAD,), jnp.float32),
        pltpu.SemaphoreType.DMA,
    ],
)
def _deg_kernel(dst_hbm, out_hbm, idx_v, ones_v, zero_v, deg_sh, sem):
    _deg_body(dst_hbm, out_hbm, idx_v, ones_v, zero_v, deg_sh, sem)


def _scatter_body(g_hbm, src_hbm, dst_hbm, zeros_hbm, out_hbm,
                  sidx_v, didx_v, rows_v, acc_sh, gsem0, gsem1,
                  ssem0, ssem1):
    cid = lax.axis_index("c")
    sid = lax.axis_index("s")
    wid = sid * NC + cid
    pltpu.sync_copy(zeros_hbm.at[pl.ds(sid * RSL, RSL)],
                    acc_sh.at[pl.ds(sid * RSL, RSL)])
    plsc.subcore_barrier()

    def gather(j, buf, sem):
        pltpu.async_copy(g_hbm.at[sidx_v.at[j]], rows_v.at[buf], sem)

    def gwait(j, buf, sem):
        pltpu.make_async_copy(g_hbm.at[sidx_v.at[j]], rows_v.at[buf],
                              sem).wait()

    def scat(j, buf, sem):
        pltpu.async_copy(rows_v.at[buf], acc_sh.at[didx_v.at[j]], sem,
                         add=True)

    def swait(j, buf, sem):
        pltpu.make_async_copy(rows_v.at[buf], acc_sh.at[didx_v.at[j]],
                              sem).wait()

    for hp in range(2):
        pltpu.sync_copy(src_hbm.at[wid, hp], sidx_v)
        pltpu.sync_copy(dst_hbm.at[wid, hp], didx_v)
        gather(0, 0, gsem0)
        gather(1, 1, gsem1)

        @pl.loop(0, (NCHH - 2) // 2)
        def _pair(p):
            j = 2 * p
            gwait(j, 0, gsem0)
            scat(j, 0, ssem0)
            swait(j, 0, ssem0)
            gather(j + 2, 0, gsem0)
            gwait(j + 1, 1, gsem1)
            scat(j + 1, 1, ssem1)
            swait(j + 1, 1, ssem1)
            gather(j + 3, 1, gsem1)

        gwait(NCHH - 2, 0, gsem0)
        scat(NCHH - 2, 0, ssem0)
        swait(NCHH - 2, 0, ssem0)
        gwait(NCHH - 1, 1, gsem1)
        scat(NCHH - 1, 1, ssem1)
        swait(NCHH - 1, 1, ssem1)

    plsc.subcore_barrier()
    pltpu.sync_copy(acc_sh.at[pl.ds(sid * RSL, RSL)],
                    out_hbm.at[cid, pl.ds(sid * RSL, RSL)])


@functools.partial(
    pl.kernel,
    out_type=jax.ShapeDtypeStruct((NC, NR, D), jnp.float32),
    mesh=_MESH,
    scratch_types=[
        pltpu.VMEM((NCHH, CH), jnp.int32),
        pltpu.VMEM((NCHH, CH), jnp.int32),
        pltpu.VMEM((2, CH, D), jnp.float32),
        pltpu.VMEM_SHARED((NR, D), jnp.float32),
        pltpu.SemaphoreType.DMA,
        pltpu.SemaphoreType.DMA,
        pltpu.SemaphoreType.DMA,
        pltpu.SemaphoreType.DMA,
    ],
)
def _scatter_kernel(g_hbm, src_hbm, dst_hbm, zeros_hbm, out_hbm,
                    sidx_v, didx_v, rows_v, acc_sh, gsem0, gsem1,
                    ssem0, ssem1):
    _scatter_body(g_hbm, src_hbm, dst_hbm, zeros_hbm, out_hbm,
                  sidx_v, didx_v, rows_v, acc_sh, gsem0, gsem1,
                  ssem0, ssem1)


RB = 2000
NRB = N_NODES // RB


def _dis(degp_ref):
    return lax.rsqrt(degp_ref[0] + degp_ref[1] + 1.0)


def _pre_body(x_ref, w_ref, degp_ref, o_ref):
    h = jnp.dot(x_ref[...], w_ref[...], preferred_element_type=jnp.float32)
    o_ref[...] = h * _dis(degp_ref)


def _pre_kernel(x, w, degp):
    return pl.pallas_call(
        _pre_body,
        grid=(NRB,),
        in_specs=[
            pl.BlockSpec((RB, D), lambda i: (i, 0)),
            pl.BlockSpec((D, D), lambda i: (0, 0)),
            pl.BlockSpec((NC, RB, 1), lambda i: (0, i, 0)),
        ],
        out_specs=pl.BlockSpec((RB, D), lambda i: (i, 0)),
        out_shape=jax.ShapeDtypeStruct((N_NODES, D), jnp.float32),
    )(x, w, degp)


def _mid_body(t_ref, g_ref, degp_ref, b_ref, w_ref, o_ref):
    dis = _dis(degp_ref)
    tsum = t_ref[0] + t_ref[1] + g_ref[...]
    h = jnp.maximum(tsum * dis + b_ref[...], 0.0)
    o_ref[...] = jnp.dot(h, w_ref[...],
                         preferred_element_type=jnp.float32) * dis


def _mid_kernel(t, g, degp, b, w):
    return pl.pallas_call(
        _mid_body,
        grid=(NRB,),
        in_specs=[
            pl.BlockSpec((NC, RB, D), lambda i: (0, i, 0)),
            pl.BlockSpec((RB, D), lambda i: (i, 0)),
            pl.BlockSpec((NC, RB, 1), lambda i: (0, i, 0)),
            pl.BlockSpec((1, D), lambda i: (0, 0)),
            pl.BlockSpec((D, D), lambda i: (0, 0)),
        ],
        out_specs=pl.BlockSpec((RB, D), lambda i: (i, 0)),
        out_shape=jax.ShapeDtypeStruct((N_NODES, D), jnp.float32),
    )(t, g, degp, b, w)


def _fin_body(t_ref, g_ref, degp_ref, b_ref, o_ref):
    tsum = t_ref[0] + t_ref[1] + g_ref[...]
    o_ref[...] = tsum * _dis(degp_ref) + b_ref[...]


def _fin_kernel(t, g, degp, b):
    return pl.pallas_call(
        _fin_body,
        grid=(NRB,),
        in_specs=[
            pl.BlockSpec((NC, RB, D), lambda i: (0, i, 0)),
            pl.BlockSpec((RB, D), lambda i: (i, 0)),
            pl.BlockSpec((NC, RB, 1), lambda i: (0, i, 0)),
            pl.BlockSpec((1, D), lambda i: (0, 0)),
        ],
        out_specs=pl.BlockSpec((RB, D), lambda i: (i, 0)),
        out_shape=jax.ShapeDtypeStruct((N_NODES, D), jnp.float32),
    )(t, g, degp, b)


def kernel(x, edge_index, W1, b1, W2, b2):
    ei = edge_index.astype(jnp.int32)
    npad = EPWP - EPW
    pad_src = jnp.broadcast_to((jnp.arange(npad, dtype=jnp.int32) * 41)
                               % N_NODES, (NW, npad))
    pad_dst = jnp.broadcast_to(N_NODES + jnp.arange(npad, dtype=jnp.int32)
                               % (NR - N_NODES), (NW, npad))
    src4 = jnp.concatenate([ei[0].reshape(NW, EPW), pad_src],
                           axis=1).reshape(NW, 2, NCHH, CH)
    dst4 = jnp.concatenate([ei[1].reshape(NW, EPW), pad_dst],
                           axis=1).reshape(NW, 2, NCHH, CH)
    dst3 = dst4.reshape(NW, NCH, CH)
    zeros = jnp.zeros((NR, D), jnp.float32)
    b1r = b1.reshape(1, D)
    b2r = b2.reshape(1, D)

    degp = _deg_kernel(dst3).reshape(NC, NPAD, 1)

    g1 = _pre_kernel(x, W1, degp)
    t1 = _scatter_kernel(g1, src4, dst4, zeros)
    g2 = _mid_kernel(t1, g1, degp, b1r, W2)
    t2 = _scatter_kernel(g2, src4, dst4, zeros)
    out = _fin_kernel(t2, g2, degp, b2r)
    return out


# speedup vs baseline: 1.1013x; 1.0001x over previous
import functools

import jax
import jax.numpy as jnp
from jax import lax
from jax.experimental import pallas as pl
from jax.experimental.pallas import tpu as pltpu
from jax.experimental.pallas import tpu_sc as plsc

N_NODES = 10000
N_EDGES = 320000
D = 128

NC = 2
NS = 16
NW = NC * NS
EPW = N_EDGES // NW
CH = 128
EPWP = 10240
NCH = EPWP // CH
NCHH = NCH // 2
NPAD = 10240
DSL = NPAD // NS
NR = 10240
RSL = NR // NS

_MESH = plsc.VectorSubcoreMesh(core_axis_name="c", subcore_axis_name="s")


def _deg_body(dst_hbm, out_hbm, idx_v, ones_v, zero_v, deg_sh, sem):
    del sem
    cid = lax.axis_index("c")
    sid = lax.axis_index("s")
    wid = sid * NC + cid
    pltpu.sync_copy(dst_hbm.at[wid], idx_v)
    for i in range(CH // 16):
        ones_v[pl.ds(i * 16, 16)] = jnp.full((16,), 1.0, jnp.float32)
    for i in range(DSL // 16):
        zero_v[pl.ds(i * 16, 16)] = jnp.zeros((16,), jnp.float32)
    pltpu.sync_copy(zero_v, deg_sh.at[pl.ds(sid * DSL, DSL)])
    plsc.subcore_barrier()

    @pl.loop(0, NCH)
    def _chunk(j):
        pltpu.sync_copy(ones_v, deg_sh.at[idx_v.at[j]], add=True)

    plsc.subcore_barrier()
    pltpu.sync_copy(deg_sh.at[pl.ds(sid * DSL, DSL)],
                    out_hbm.at[cid, pl.ds(sid * DSL, DSL)])


@functools.partial(
    pl.kernel,
    out_type=jax.ShapeDtypeStruct((NC, NPAD), jnp.float32),
    mesh=_MESH,
    scratch_types=[
        pltpu.VMEM((NCH, CH), jnp.int32),
        pltpu.VMEM((CH,), jnp.float32),
        pltpu.VMEM((DSL,), jnp.float32),
        pltpu.VMEM_SHARED((NPAD,), jnp.float32),
        pltpu.SemaphoreType.DMA,
    ],
)
def _deg_kernel(dst_hbm, out_hbm, idx_v, ones_v, zero_v, deg_sh, sem):
    _deg_body(dst_hbm, out_hbm, idx_v, ones_v, zero_v, deg_sh, sem)


def _scatter_body(g_hbm, src_hbm, dst_hbm, zeros_hbm, out_hbm,
                  sidx_v, didx_v, rows_v, acc_sh, gsem0, gsem1,
                  ssem0, ssem1):
    cid = lax.axis_index("c")
    sid = lax.axis_index("s")
    wid = sid * NC + cid
    pltpu.sync_copy(zeros_hbm.at[pl.ds(sid * RSL, RSL)],
                    acc_sh.at[pl.ds(sid * RSL, RSL)])
    plsc.subcore_barrier()

    def gather(j, buf, sem):
        pltpu.async_copy(g_hbm.at[sidx_v.at[j]], rows_v.at[buf], sem)

    def gwait(j, buf, sem):
        pltpu.make_async_copy(g_hbm.at[sidx_v.at[j]], rows_v.at[buf],
                              sem).wait()

    def scat(j, buf, sem):
        pltpu.async_copy(rows_v.at[buf], acc_sh.at[didx_v.at[j]], sem,
                         add=True)

    def swait(j, buf, sem):
        pltpu.make_async_copy(rows_v.at[buf], acc_sh.at[didx_v.at[j]],
                              sem).wait()

    for hp in range(2):
        pltpu.sync_copy(src_hbm.at[wid, hp], sidx_v)
        pltpu.sync_copy(dst_hbm.at[wid, hp], didx_v)
        gather(0, 0, gsem0)
        gather(1, 1, gsem1)

        @pl.loop(0, (NCHH - 2) // 2)
        def _pair(p):
            j = 2 * p
            gwait(j, 0, gsem0)
            scat(j, 0, ssem0)
            swait(j, 0, ssem0)
            gather(j + 2, 0, gsem0)
            gwait(j + 1, 1, gsem1)
            scat(j + 1, 1, ssem1)
            swait(j + 1, 1, ssem1)
            gather(j + 3, 1, gsem1)

        gwait(NCHH - 2, 0, gsem0)
        scat(NCHH - 2, 0, ssem0)
        swait(NCHH - 2, 0, ssem0)
        gwait(NCHH - 1, 1, gsem1)
        scat(NCHH - 1, 1, ssem1)
        swait(NCHH - 1, 1, ssem1)

    plsc.subcore_barrier()
    pltpu.sync_copy(acc_sh.at[pl.ds(sid * RSL, RSL)],
                    out_hbm.at[cid, pl.ds(sid * RSL, RSL)])


@functools.partial(
    pl.kernel,
    out_type=jax.ShapeDtypeStruct((NC, NR, D), jnp.float32),
    mesh=_MESH,
    scratch_types=[
        pltpu.VMEM((NCHH, CH), jnp.int32),
        pltpu.VMEM((NCHH, CH), jnp.int32),
        pltpu.VMEM((2, CH, D), jnp.float32),
        pltpu.VMEM_SHARED((NR, D), jnp.float32),
        pltpu.SemaphoreType.DMA,
        pltpu.SemaphoreType.DMA,
        pltpu.SemaphoreType.DMA,
        pltpu.SemaphoreType.DMA,
    ],
)
def _scatter_kernel(g_hbm, src_hbm, dst_hbm, zeros_hbm, out_hbm,
                    sidx_v, didx_v, rows_v, acc_sh, gsem0, gsem1,
                    ssem0, ssem1):
    _scatter_body(g_hbm, src_hbm, dst_hbm, zeros_hbm, out_hbm,
                  sidx_v, didx_v, rows_v, acc_sh, gsem0, gsem1,
                  ssem0, ssem1)


RB = 5000
NRB = N_NODES // RB


def _dis(degp_ref):
    return lax.rsqrt(degp_ref[0] + degp_ref[1] + 1.0)


def _pre_body(x_ref, w_ref, degp_ref, o_ref):
    h = jnp.dot(x_ref[...], w_ref[...], preferred_element_type=jnp.float32)
    o_ref[...] = h * _dis(degp_ref)


def _pre_kernel(x, w, degp):
    return pl.pallas_call(
        _pre_body,
        grid=(NRB,),
        in_specs=[
            pl.BlockSpec((RB, D), lambda i: (i, 0)),
            pl.BlockSpec((D, D), lambda i: (0, 0)),
            pl.BlockSpec((NC, RB, 1), lambda i: (0, i, 0)),
        ],
        out_specs=pl.BlockSpec((RB, D), lambda i: (i, 0)),
        out_shape=jax.ShapeDtypeStruct((N_NODES, D), jnp.float32),
    )(x, w, degp)


def _mid_body(t_ref, g_ref, degp_ref, b_ref, w_ref, o_ref):
    dis = _dis(degp_ref)
    tsum = t_ref[0] + t_ref[1] + g_ref[...]
    h = jnp.maximum(tsum * dis + b_ref[...], 0.0)
    o_ref[...] = jnp.dot(h, w_ref[...],
                         preferred_element_type=jnp.float32) * dis


def _mid_kernel(t, g, degp, b, w):
    return pl.pallas_call(
        _mid_body,
        grid=(NRB,),
        in_specs=[
            pl.BlockSpec((NC, RB, D), lambda i: (0, i, 0)),
            pl.BlockSpec((RB, D), lambda i: (i, 0)),
            pl.BlockSpec((NC, RB, 1), lambda i: (0, i, 0)),
            pl.BlockSpec((1, D), lambda i: (0, 0)),
            pl.BlockSpec((D, D), lambda i: (0, 0)),
        ],
        out_specs=pl.BlockSpec((RB, D), lambda i: (i, 0)),
        out_shape=jax.ShapeDtypeStruct((N_NODES, D), jnp.float32),
    )(t, g, degp, b, w)


def _fin_body(t_ref, g_ref, degp_ref, b_ref, o_ref):
    tsum = t_ref[0] + t_ref[1] + g_ref[...]
    o_ref[...] = tsum * _dis(degp_ref) + b_ref[...]


def _fin_kernel(t, g, degp, b):
    return pl.pallas_call(
        _fin_body,
        grid=(NRB,),
        in_specs=[
            pl.BlockSpec((NC, RB, D), lambda i: (0, i, 0)),
            pl.BlockSpec((RB, D), lambda i: (i, 0)),
            pl.BlockSpec((NC, RB, 1), lambda i: (0, i, 0)),
            pl.BlockSpec((1, D), lambda i: (0, 0)),
        ],
        out_specs=pl.BlockSpec((RB, D), lambda i: (i, 0)),
        out_shape=jax.ShapeDtypeStruct((N_NODES, D), jnp.float32),
    )(t, g, degp, b)


def kernel(x, edge_index, W1, b1, W2, b2):
    ei = edge_index.astype(jnp.int32)
    npad = EPWP - EPW
    pad_src = jnp.broadcast_to((jnp.arange(npad, dtype=jnp.int32) * 41)
                               % N_NODES, (NW, npad))
    pad_dst = jnp.broadcast_to(N_NODES + jnp.arange(npad, dtype=jnp.int32)
                               % (NR - N_NODES), (NW, npad))
    src4 = jnp.concatenate([ei[0].reshape(NW, EPW), pad_src],
                           axis=1).reshape(NW, 2, NCHH, CH)
    dst4 = jnp.concatenate([ei[1].reshape(NW, EPW), pad_dst],
                           axis=1).reshape(NW, 2, NCHH, CH)
    dst3 = dst4.reshape(NW, NCH, CH)
    zeros = jnp.zeros((NR, D), jnp.float32)
    b1r = b1.reshape(1, D)
    b2r = b2.reshape(1, D)

    degp = _deg_kernel(dst3).reshape(NC, NPAD, 1)

    g1 = _pre_kernel(x, W1, degp)
    t1 = _scatter_kernel(g1, src4, dst4, zeros)
    g2 = _mid_kernel(t1, g1, degp, b1r, W2)
    t2 = _scatter_kernel(g2, src4, dst4, zeros)
    out = _fin_kernel(t2, g2, degp, b2r)
    return out


# speedup vs baseline: 1.1298x; 1.0259x over previous
import functools

import jax
import jax.numpy as jnp
from jax import lax
from jax.experimental import pallas as pl
from jax.experimental.pallas import tpu as pltpu
from jax.experimental.pallas import tpu_sc as plsc

N_NODES = 10000
N_EDGES = 320000
D = 128

NC = 2
NS = 16
NW = NC * NS
EPW = N_EDGES // NW
CH = 128
EPWP = 10240
NCH = EPWP // CH
NCHH = NCH // 2
NPAD = 10240
DSL = NPAD // NS
NR = 10240
RSL = NR // NS

_MESH = plsc.VectorSubcoreMesh(core_axis_name="c", subcore_axis_name="s")


def _deg_body(dst_hbm, out_hbm, idx_v, ones_v, zero_v, deg_sh, sem):
    del sem
    cid = lax.axis_index("c")
    sid = lax.axis_index("s")
    wid = sid * NC + cid
    pltpu.sync_copy(dst_hbm.at[wid], idx_v)
    for i in range(CH // 16):
        ones_v[pl.ds(i * 16, 16)] = jnp.full((16,), 1.0, jnp.float32)
    for i in range(DSL // 16):
        zero_v[pl.ds(i * 16, 16)] = jnp.zeros((16,), jnp.float32)
    pltpu.sync_copy(zero_v, deg_sh.at[pl.ds(sid * DSL, DSL)])
    plsc.subcore_barrier()

    @pl.loop(0, NCH)
    def _chunk(j):
        pltpu.sync_copy(ones_v, deg_sh.at[idx_v.at[j]], add=True)

    plsc.subcore_barrier()
    pltpu.sync_copy(deg_sh.at[pl.ds(sid * DSL, DSL)],
                    out_hbm.at[cid, pl.ds(sid * DSL, DSL)])


@functools.partial(
    pl.kernel,
    out_type=jax.ShapeDtypeStruct((NC, NPAD), jnp.float32),
    mesh=_MESH,
    scratch_types=[
        pltpu.VMEM((NCH, CH), jnp.int32),
        pltpu.VMEM((CH,), jnp.float32),
        pltpu.VMEM((DSL,), jnp.float32),
        pltpu.VMEM_SHARED((NPAD,), jnp.float32),
        pltpu.SemaphoreType.DMA,
    ],
)
def _deg_kernel(dst_hbm, out_hbm, idx_v, ones_v, zero_v, deg_sh, sem):
    _deg_body(dst_hbm, out_hbm, idx_v, ones_v, zero_v, deg_sh, sem)


def _scatter_body(g_hbm, src_hbm, dst_hbm, out_hbm,
                  sidx_v, didx_v, rows_v, acc_sh, gsem0, gsem1,
                  ssem0, ssem1):
    cid = lax.axis_index("c")
    sid = lax.axis_index("s")
    wid = sid * NC + cid

    @pl.loop(0, CH)
    def _zrow(i):
        for k in range(D // 16):
            rows_v[0, i, pl.ds(k * 16, 16)] = jnp.zeros((16,), jnp.float32)

    for r in range(RSL // CH):
        pltpu.sync_copy(rows_v.at[0],
                        acc_sh.at[pl.ds(sid * RSL + r * CH, CH)])
    plsc.subcore_barrier()

    def gather(j, buf, sem):
        pltpu.async_copy(g_hbm.at[sidx_v.at[j]], rows_v.at[buf], sem)

    def gwait(j, buf, sem):
        pltpu.make_async_copy(g_hbm.at[sidx_v.at[j]], rows_v.at[buf],
                              sem).wait()

    def scat(j, buf, sem):
        pltpu.async_copy(rows_v.at[buf], acc_sh.at[didx_v.at[j]], sem,
                         add=True)

    def swait(j, buf, sem):
        pltpu.make_async_copy(rows_v.at[buf], acc_sh.at[didx_v.at[j]],
                              sem).wait()

    for hp in range(2):
        pltpu.sync_copy(src_hbm.at[wid, hp], sidx_v)
        pltpu.sync_copy(dst_hbm.at[wid, hp], didx_v)
        gather(0, 0, gsem0)
        gather(1, 1, gsem1)

        @pl.loop(0, (NCHH - 2) // 2)
        def _pair(p):
            j = 2 * p
            gwait(j, 0, gsem0)
            scat(j, 0, ssem0)
            swait(j, 0, ssem0)
            gather(j + 2, 0, gsem0)
            gwait(j + 1, 1, gsem1)
            scat(j + 1, 1, ssem1)
            swait(j + 1, 1, ssem1)
            gather(j + 3, 1, gsem1)

        gwait(NCHH - 2, 0, gsem0)
        scat(NCHH - 2, 0, ssem0)
        swait(NCHH - 2, 0, ssem0)
        gwait(NCHH - 1, 1, gsem1)
        scat(NCHH - 1, 1, ssem1)
        swait(NCHH - 1, 1, ssem1)

    plsc.subcore_barrier()
    pltpu.sync_copy(acc_sh.at[pl.ds(sid * RSL, RSL)],
                    out_hbm.at[cid, pl.ds(sid * RSL, RSL)])


@functools.partial(
    pl.kernel,
    out_type=jax.ShapeDtypeStruct((NC, NR, D), jnp.float32),
    mesh=_MESH,
    scratch_types=[
        pltpu.VMEM((NCHH, CH), jnp.int32),
        pltpu.VMEM((NCHH, CH), jnp.int32),
        pltpu.VMEM((2, CH, D), jnp.float32),
        pltpu.VMEM_SHARED((NR, D), jnp.float32),
        pltpu.SemaphoreType.DMA,
        pltpu.SemaphoreType.DMA,
        pltpu.SemaphoreType.DMA,
        pltpu.SemaphoreType.DMA,
    ],
)
def _scatter_kernel(g_hbm, src_hbm, dst_hbm, out_hbm,
                    sidx_v, didx_v, rows_v, acc_sh, gsem0, gsem1,
                    ssem0, ssem1):
    _scatter_body(g_hbm, src_hbm, dst_hbm, out_hbm,
                  sidx_v, didx_v, rows_v, acc_sh, gsem0, gsem1,
                  ssem0, ssem1)


RB = 2000
NRB = N_NODES // RB


def _dis(degp_ref):
    return lax.rsqrt(degp_ref[0] + degp_ref[1] + 1.0)


def _pre_body(x_ref, w_ref, degp_ref, o_ref):
    h = jnp.dot(x_ref[...], w_ref[...], preferred_element_type=jnp.float32)
    o_ref[...] = h * _dis(degp_ref)


def _pre_kernel(x, w, degp):
    return pl.pallas_call(
        _pre_body,
        grid=(NRB,),
        in_specs=[
            pl.BlockSpec((RB, D), lambda i: (i, 0)),
            pl.BlockSpec((D, D), lambda i: (0, 0)),
            pl.BlockSpec((NC, RB, 1), lambda i: (0, i, 0)),
        ],
        out_specs=pl.BlockSpec((RB, D), lambda i: (i, 0)),
        out_shape=jax.ShapeDtypeStruct((N_NODES, D), jnp.float32),
    )(x, w, degp)


def _mid_body(t_ref, g_ref, degp_ref, b_ref, w_ref, o_ref):
    dis = _dis(degp_ref)
    tsum = t_ref[0] + t_ref[1] + g_ref[...]
    h = jnp.maximum(tsum * dis + b_ref[...], 0.0)
    o_ref[...] = jnp.dot(h, w_ref[...],
                         preferred_element_type=jnp.float32) * dis


def _mid_kernel(t, g, degp, b, w):
    return pl.pallas_call(
        _mid_body,
        grid=(NRB,),
        in_specs=[
            pl.BlockSpec((NC, RB, D), lambda i: (0, i, 0)),
            pl.BlockSpec((RB, D), lambda i: (i, 0)),
            pl.BlockSpec((NC, RB, 1), lambda i: (0, i, 0)),
            pl.BlockSpec((1, D), lambda i: (0, 0)),
            pl.BlockSpec((D, D), lambda i: (0, 0)),
        ],
        out_specs=pl.BlockSpec((RB, D), lambda i: (i, 0)),
        out_shape=jax.ShapeDtypeStruct((N_NODES, D), jnp.float32),
    )(t, g, degp, b, w)


def _fin_body(t_ref, g_ref, degp_ref, b_ref, o_ref):
    tsum = t_ref[0] + t_ref[1] + g_ref[...]
    o_ref[...] = tsum * _dis(degp_ref) + b_ref[...]


def _fin_kernel(t, g, degp, b):
    return pl.pallas_call(
        _fin_body,
        grid=(NRB,),
        in_specs=[
            pl.BlockSpec((NC, RB, D), lambda i: (0, i, 0)),
            pl.BlockSpec((RB, D), lambda i: (i, 0)),
            pl.BlockSpec((NC, RB, 1), lambda i: (0, i, 0)),
            pl.BlockSpec((1, D), lambda i: (0, 0)),
        ],
        out_specs=pl.BlockSpec((RB, D), lambda i: (i, 0)),
        out_shape=jax.ShapeDtypeStruct((N_NODES, D), jnp.float32),
    )(t, g, degp, b)


def kernel(x, edge_index, W1, b1, W2, b2):
    ei = edge_index.astype(jnp.int32)
    npad = EPWP - EPW
    pad_src = jnp.broadcast_to((jnp.arange(npad, dtype=jnp.int32) * 41)
                               % N_NODES, (NW, npad))
    pad_dst = jnp.broadcast_to(N_NODES + jnp.arange(npad, dtype=jnp.int32)
                               % (NR - N_NODES), (NW, npad))
    src4 = jnp.concatenate([ei[0].reshape(NW, EPW), pad_src],
                           axis=1).reshape(NW, 2, NCHH, CH)
    dst4 = jnp.concatenate([ei[1].reshape(NW, EPW), pad_dst],
                           axis=1).reshape(NW, 2, NCHH, CH)
    dst3 = dst4.reshape(NW, NCH, CH)
    b1r = b1.reshape(1, D)
    b2r = b2.reshape(1, D)

    degp = _deg_kernel(dst3).reshape(NC, NPAD, 1)

    g1 = _pre_kernel(x, W1, degp)
    t1 = _scatter_kernel(g1, src4, dst4)
    g2 = _mid_kernel(t1, g1, degp, b1r, W2)
    t2 = _scatter_kernel(g2, src4, dst4)
    out = _fin_kernel(t2, g2, degp, b2r)
    return out


# speedup vs baseline: 1.1496x; 1.0175x over previous
import functools

import jax
import jax.numpy as jnp
from jax import lax
from jax.experimental import pallas as pl
from jax.experimental.pallas import tpu as pltpu
from jax.experimental.pallas import tpu_sc as plsc

N_NODES = 10000
N_EDGES = 320000
D = 128

NC = 2
NS = 16
NW = NC * NS
EPW = N_EDGES // NW
CH = 128
EPWP = 10240
NCH = EPWP // CH
NCHH = NCH // 2
NPAD = 10240
DSL = NPAD // NS
NR = 10240
RSL = NR // NS

_MESH = plsc.VectorSubcoreMesh(core_axis_name="c", subcore_axis_name="s")


def _deg_body(dst_hbm, out_hbm, idx_v, ones_v, zero_v, deg_sh, sem):
    cid = lax.axis_index("c")
    sid = lax.axis_index("s")
    wid = sid * NC + cid
    pltpu.sync_copy(dst_hbm.at[wid], idx_v)
    for i in range(CH // 16):
        ones_v[pl.ds(i * 16, 16)] = jnp.full((16,), 1.0, jnp.float32)
    for i in range(DSL // 16):
        zero_v[pl.ds(i * 16, 16)] = jnp.zeros((16,), jnp.float32)
    pltpu.sync_copy(zero_v, deg_sh.at[pl.ds(sid * DSL, DSL)])
    plsc.subcore_barrier()

    @pl.loop(0, NCH)
    def _chunk(j):
        pltpu.async_copy(ones_v, deg_sh.at[idx_v.at[j]], sem, add=True)

    @pl.loop(0, NCH)
    def _drain(j):
        pltpu.make_async_copy(ones_v, deg_sh.at[idx_v.at[j]], sem).wait()

    plsc.subcore_barrier()
    pltpu.sync_copy(deg_sh.at[pl.ds(sid * DSL, DSL)],
                    out_hbm.at[cid, pl.ds(sid * DSL, DSL)])


@functools.partial(
    pl.kernel,
    out_type=jax.ShapeDtypeStruct((NC, NPAD), jnp.float32),
    mesh=_MESH,
    scratch_types=[
        pltpu.VMEM((NCH, CH), jnp.int32),
        pltpu.VMEM((CH,), jnp.float32),
        pltpu.VMEM((DSL,), jnp.float32),
        pltpu.VMEM_SHARED((NPAD,), jnp.float32),
        pltpu.SemaphoreType.DMA,
    ],
)
def _deg_kernel(dst_hbm, out_hbm, idx_v, ones_v, zero_v, deg_sh, sem):
    _deg_body(dst_hbm, out_hbm, idx_v, ones_v, zero_v, deg_sh, sem)


def _scatter_body(g_hbm, src_hbm, dst_hbm, out_hbm,
                  sidx_v, didx_v, rows_v, acc_sh, gsem0, gsem1,
                  ssem0, ssem1):
    cid = lax.axis_index("c")
    sid = lax.axis_index("s")
    wid = sid * NC + cid

    @pl.loop(0, CH)
    def _zrow(i):
        for k in range(D // 16):
            rows_v[0, i, pl.ds(k * 16, 16)] = jnp.zeros((16,), jnp.float32)

    for r in range(RSL // CH):
        pltpu.sync_copy(rows_v.at[0],
                        acc_sh.at[pl.ds(sid * RSL + r * CH, CH)])
    plsc.subcore_barrier()

    def gather(j, buf, sem):
        pltpu.async_copy(g_hbm.at[sidx_v.at[j]], rows_v.at[buf], sem)

    def gwait(j, buf, sem):
        pltpu.make_async_copy(g_hbm.at[sidx_v.at[j]], rows_v.at[buf],
                              sem).wait()

    def scat(j, buf, sem):
        pltpu.async_copy(rows_v.at[buf], acc_sh.at[didx_v.at[j]], sem,
                         add=True)

    def swait(j, buf, sem):
        pltpu.make_async_copy(rows_v.at[buf], acc_sh.at[didx_v.at[j]],
                              sem).wait()

    for hp in range(2):
        pltpu.sync_copy(src_hbm.at[wid, hp], sidx_v)
        pltpu.sync_copy(dst_hbm.at[wid, hp], didx_v)
        gather(0, 0, gsem0)
        gather(1, 1, gsem1)

        @pl.loop(0, (NCHH - 2) // 2)
        def _pair(p):
            j = 2 * p
            gwait(j, 0, gsem0)
            scat(j, 0, ssem0)
            swait(j, 0, ssem0)
            gather(j + 2, 0, gsem0)
            gwait(j + 1, 1, gsem1)
            scat(j + 1, 1, ssem1)
            swait(j + 1, 1, ssem1)
            gather(j + 3, 1, gsem1)

        gwait(NCHH - 2, 0, gsem0)
        scat(NCHH - 2, 0, ssem0)
        swait(NCHH - 2, 0, ssem0)
        gwait(NCHH - 1, 1, gsem1)
        scat(NCHH - 1, 1, ssem1)
        swait(NCHH - 1, 1, ssem1)

    plsc.subcore_barrier()
    pltpu.sync_copy(acc_sh.at[pl.ds(sid * RSL, RSL)],
                    out_hbm.at[cid, pl.ds(sid * RSL, RSL)])


@functools.partial(
    pl.kernel,
    out_type=jax.ShapeDtypeStruct((NC, NR, D), jnp.float32),
    mesh=_MESH,
    scratch_types=[
        pltpu.VMEM((NCHH, CH), jnp.int32),
        pltpu.VMEM((NCHH, CH), jnp.int32),
        pltpu.VMEM((2, CH, D), jnp.float32),
        pltpu.VMEM_SHARED((NR, D), jnp.float32),
        pltpu.SemaphoreType.DMA,
        pltpu.SemaphoreType.DMA,
        pltpu.SemaphoreType.DMA,
        pltpu.SemaphoreType.DMA,
    ],
)
def _scatter_kernel(g_hbm, src_hbm, dst_hbm, out_hbm,
                    sidx_v, didx_v, rows_v, acc_sh, gsem0, gsem1,
                    ssem0, ssem1):
    _scatter_body(g_hbm, src_hbm, dst_hbm, out_hbm,
                  sidx_v, didx_v, rows_v, acc_sh, gsem0, gsem1,
                  ssem0, ssem1)


RB = 2000
NRB = N_NODES // RB


def _dis(degp_ref):
    return lax.rsqrt(degp_ref[0] + degp_ref[1] + 1.0)


def _pre_body(x_ref, w_ref, degp_ref, o_ref):
    h = jnp.dot(x_ref[...], w_ref[...], preferred_element_type=jnp.float32)
    o_ref[...] = h * _dis(degp_ref)


def _pre_kernel(x, w, degp):
    return pl.pallas_call(
        _pre_body,
        grid=(NRB,),
        in_specs=[
            pl.BlockSpec((RB, D), lambda i: (i, 0)),
            pl.BlockSpec((D, D), lambda i: (0, 0)),
            pl.BlockSpec((NC, RB, 1), lambda i: (0, i, 0)),
        ],
        out_specs=pl.BlockSpec((RB, D), lambda i: (i, 0)),
        out_shape=jax.ShapeDtypeStruct((N_NODES, D), jnp.float32),
    )(x, w, degp)


def _mid_body(t_ref, g_ref, degp_ref, b_ref, w_ref, o_ref):
    dis = _dis(degp_ref)
    tsum = t_ref[0] + t_ref[1] + g_ref[...]
    h = jnp.maximum(tsum * dis + b_ref[...], 0.0)
    o_ref[...] = jnp.dot(h, w_ref[...],
                         preferred_element_type=jnp.float32) * dis


def _mid_kernel(t, g, degp, b, w):
    return pl.pallas_call(
        _mid_body,
        grid=(NRB,),
        in_specs=[
            pl.BlockSpec((NC, RB, D), lambda i: (0, i, 0)),
            pl.BlockSpec((RB, D), lambda i: (i, 0)),
            pl.BlockSpec((NC, RB, 1), lambda i: (0, i, 0)),
            pl.BlockSpec((1, D), lambda i: (0, 0)),
            pl.BlockSpec((D, D), lambda i: (0, 0)),
        ],
        out_specs=pl.BlockSpec((RB, D), lambda i: (i, 0)),
        out_shape=jax.ShapeDtypeStruct((N_NODES, D), jnp.float32),
    )(t, g, degp, b, w)


def _fin_body(t_ref, g_ref, degp_ref, b_ref, o_ref):
    tsum = t_ref[0] + t_ref[1] + g_ref[...]
    o_ref[...] = tsum * _dis(degp_ref) + b_ref[...]


def _fin_kernel(t, g, degp, b):
    return pl.pallas_call(
        _fin_body,
        grid=(NRB,),
        in_specs=[
            pl.BlockSpec((NC, RB, D), lambda i: (0, i, 0)),
            pl.BlockSpec((RB, D), lambda i: (i, 0)),
            pl.BlockSpec((NC, RB, 1), lambda i: (0, i, 0)),
            pl.BlockSpec((1, D), lambda i: (0, 0)),
        ],
        out_specs=pl.BlockSpec((RB, D), lambda i: (i, 0)),
        out_shape=jax.ShapeDtypeStruct((N_NODES, D), jnp.float32),
    )(t, g, degp, b)


def kernel(x, edge_index, W1, b1, W2, b2):
    ei = edge_index.astype(jnp.int32)
    npad = EPWP - EPW
    pad_src = jnp.broadcast_to((jnp.arange(npad, dtype=jnp.int32) * 41)
                               % N_NODES, (NW, npad))
    pad_dst = jnp.broadcast_to(N_NODES + jnp.arange(npad, dtype=jnp.int32)
                               % (NR - N_NODES), (NW, npad))
    src4 = jnp.concatenate([ei[0].reshape(NW, EPW), pad_src],
                           axis=1).reshape(NW, 2, NCHH, CH)
    dst4 = jnp.concatenate([ei[1].reshape(NW, EPW), pad_dst],
                           axis=1).reshape(NW, 2, NCHH, CH)
    dst3 = dst4.reshape(NW, NCH, CH)
    b1r = b1.reshape(1, D)
    b2r = b2.reshape(1, D)

    degp = _deg_kernel(dst3).reshape(NC, NPAD, 1)

    g1 = _pre_kernel(x, W1, degp)
    t1 = _scatter_kernel(g1, src4, dst4)
    g2 = _mid_kernel(t1, g1, degp, b1r, W2)
    t2 = _scatter_kernel(g2, src4, dst4)
    out = _fin_kernel(t2, g2, degp, b2r)
    return out
